# baseline (speedup 1.0000x reference)
import functools

import jax
import jax.numpy as jnp
from jax import lax
from jax.experimental import pallas as pl
from jax.experimental.pallas import tpu as pltpu
from jax.experimental.pallas import tpu_sc as plsc

NC = 2
NS = 16
CHUNK = 80
VBUF = 5
IBUF = 10
HZN = 2
IO_CH = 640


def _make_sc_spmm(n, e, d):
    dh = d // 2
    ept = e // NS
    nch = ept // CHUNK
    nround = nch // IBUF
    assert ept * NS == e and nch * CHUNK == ept and nround * IBUF == nch
    look = VBUF - HZN

    mesh = plsc.VectorSubcoreMesh(
        core_axis_name="c", subcore_axis_name="s", num_cores=NC,
        num_subcores=NS)

    def body(rc_h, embA_h, embB_h, ones_h, zerd_h, zer1_h,
             f_out, g2_out, cnt_out, o2_out,
             f_acc, g_acc, cnt_acc, o2_acc,
             ibuf, vec, sbuf, onesbuf,
             semi, semg, sems, semsg, semss):
        c = lax.axis_index("c")
        s = lax.axis_index("s")
        ibase = jnp.minimum(IO_CH * s, n - IO_CH)

        pltpu.sync_copy(zerd_h, f_acc.at[pl.ds(ibase, IO_CH), :])
        pltpu.sync_copy(zerd_h, g_acc.at[pl.ds(ibase, IO_CH), :])

        @pl.when(c == 0)
        def _():
            pltpu.sync_copy(zer1_h, cnt_acc.at[pl.ds(ibase, IO_CH)])
            pltpu.sync_copy(zer1_h, o2_acc.at[pl.ds(ibase, IO_CH)])

        pltpu.sync_copy(ones_h, onesbuf)
        plsc.subcore_barrier()

        def istart(k, slot):
            pltpu.async_copy(rc_h.at[s * nch + k], ibuf.at[slot],
                             semi.at[slot])

        def iwait(slot):
            pltpu.make_async_copy(rc_h.at[0], ibuf.at[slot],
                                  semi.at[slot]).wait()

        def run_pass(tbl, acc, scalar_tbl, scalar_acc, scalar_ones):
            do_scalar = scalar_acc is not None

            def gstart(k, bv, bi):
                pltpu.async_copy(tbl.at[ibuf.at[bi, 1]], vec.at[bv],
                                 semg.at[bv])
                if do_scalar and not scalar_ones:
                    pltpu.async_copy(scalar_tbl.at[ibuf.at[bi, 1]],
                                     sbuf.at[bv], semsg.at[bv])

            def gwait(bv):
                pltpu.make_async_copy(tbl.at[ibuf.at[0, 1]], vec.at[bv],
                                      semg.at[bv]).wait()
                if do_scalar and not scalar_ones:
                    pltpu.make_async_copy(scalar_tbl.at[ibuf.at[0, 1]],
                                          sbuf.at[bv], semsg.at[bv]).wait()

            def sstart(k, bv, bi):
                pltpu.async_copy(vec.at[bv], acc.at[ibuf.at[bi, 0]],
                                 sems.at[bv], add=True)
                if do_scalar:
                    src = onesbuf if scalar_ones else sbuf.at[bv]
                    pltpu.async_copy(src, scalar_acc.at[ibuf.at[bi, 0]],
                                     semss.at[bv], add=True)

            def swait(bv):
                pltpu.make_async_copy(vec.at[bv], acc.at[ibuf.at[0, 0]],
                                      sems.at[bv]).wait()
                if do_scalar:
                    src = onesbuf if scalar_ones else sbuf.at[bv]
                    pltpu.make_async_copy(src, scalar_acc.at[ibuf.at[0, 0]],
                                          semss.at[bv]).wait()

            for m in range(IBUF):
                istart(m, m)
            for b in range(VBUF):
                iwait(b)
                gstart(b, b, b)

            def round_body(g, carry):
                for m in range(IBUF):
                    k = g * IBUF + m
                    bv = m % VBUF
                    gwait(bv)
                    sstart(k, bv, m)
                    j = k - HZN
                    bj = (m - HZN) % VBUF
                    ij = (m - HZN) % IBUF

                    @pl.when(j >= 0)
                    def _():
                        swait(bj)

                        @pl.when(k + look < nch)
                        def _():
                            iwait((m + look) % IBUF)
                            gstart(k + look, bj, (m + look) % IBUF)

                        @pl.when(j + IBUF < nch)
                        def _():
                            istart(j + IBUF, ij)

                return carry

            lax.fori_loop(0, nround, round_body, 0)
            for t in range(HZN):
                swait((nch - HZN + t) % VBUF)

        @pl.when(c == 0)
        def _():
            run_pass(embA_h, f_acc, None, cnt_acc, True)

        @pl.when(c == 1)
        def _():
            run_pass(embB_h, f_acc, None, None, False)

        plsc.subcore_barrier()

        @pl.when(c == 0)
        def _():
            run_pass(f_acc, g_acc, cnt_acc, o2_acc, False)

        @pl.when(c == 1)
        def _():
            run_pass(f_acc, g_acc, None, None, False)

        plsc.subcore_barrier()

        pltpu.sync_copy(f_acc.at[pl.ds(ibase, IO_CH), :],
                        f_out.at[pl.ds(ibase, IO_CH), pl.ds(dh * c, dh)])
        pltpu.sync_copy(g_acc.at[pl.ds(ibase, IO_CH), :],
                        g2_out.at[pl.ds(ibase, IO_CH), pl.ds(dh * c, dh)])

        @pl.when(c == 0)
        def _():
            pltpu.sync_copy(cnt_acc.at[pl.ds(ibase, IO_CH)],
                            cnt_out.at[pl.ds(ibase, IO_CH)])
            pltpu.sync_copy(o2_acc.at[pl.ds(ibase, IO_CH)],
                            o2_out.at[pl.ds(ibase, IO_CH)])

    f32 = jnp.float32
    return pl.kernel(
        body,
        out_type=(
            jax.ShapeDtypeStruct((n, d), f32),
            jax.ShapeDtypeStruct((n, d), f32),
            jax.ShapeDtypeStruct((n,), f32),
            jax.ShapeDtypeStruct((n,), f32),
        ),
        mesh=mesh,
        compiler_params=pltpu.CompilerParams(use_tc_tiling_on_sc=False),
        scratch_types=[
            pltpu.VMEM_SHARED((n, dh), f32),
            pltpu.VMEM_SHARED((n, dh), f32),
            pltpu.VMEM_SHARED((n,), f32),
            pltpu.VMEM_SHARED((n,), f32),
            pltpu.VMEM((IBUF, 2, CHUNK), jnp.int32),
            pltpu.VMEM((VBUF, CHUNK, dh), f32),
            pltpu.VMEM((VBUF, CHUNK), f32),
            pltpu.VMEM((CHUNK,), f32),
            pltpu.SemaphoreType.DMA((IBUF,)),
            pltpu.SemaphoreType.DMA((VBUF,)),
            pltpu.SemaphoreType.DMA((VBUF,)),
            pltpu.SemaphoreType.DMA((VBUF,)),
            pltpu.SemaphoreType.DMA((VBUF,)),
        ],
    )


def _score_body(e_ref, y1_ref, y2_ref, c_ref, q_ref, u_ref, out_ref):
    emb = e_ref[...]
    y1 = y1_ref[...]
    y2 = y2_ref[...]
    c = c_ref[...]
    q = q_ref[...]
    u = u_ref[...]
    f = y1 - emb
    scd = (y2 - y1) - f - c * emb
    num = f + scd
    scd_num = q - c - c
    den = c + scd_num + 1e-8
    sub = num / den
    nrm = jnp.sqrt(jnp.sum(sub * sub, axis=-1, keepdims=True))
    sub = sub / jnp.maximum(nrm, 1e-12)
    enrm = jnp.sqrt(jnp.sum(emb * emb, axis=-1, keepdims=True))
    embn = emb / jnp.maximum(enrm, 1e-12)
    dot = jnp.sum(sub * embn, axis=-1, keepdims=True)
    sig = jax.nn.sigmoid(dot)
    uu = jnp.where(u == 0, 1e-8, u)
    noise = -jnp.log(-jnp.log(uu))
    out_ref[...] = jnp.log(sig) + noise


def _score_call(emb, f, g2, cnt, o2, u):
    n, d = emb.shape
    br = 2000
    grid = (n // br,)
    rspec = pl.BlockSpec((br, d), lambda i: (i, 0))
    cspec = pl.BlockSpec((br, 1), lambda i: (i, 0))
    return pl.pallas_call(
        _score_body,
        grid=grid,
        in_specs=[rspec, rspec, rspec, cspec, cspec, cspec],
        out_specs=cspec,
        out_shape=jax.ShapeDtypeStruct((n, 1), jnp.float32),
    )(emb, f, g2, cnt.reshape(n, 1), o2.reshape(n, 1), u.reshape(n, 1))


def kernel(allOneAdj, embeds):
    n, d = embeds.shape
    e = allOneAdj.shape[1]
    dh = d // 2
    rc = jnp.stack([allOneAdj[0].reshape(e // CHUNK, CHUNK),
                    allOneAdj[1].reshape(e // CHUNK, CHUNK)], axis=1)
    embA = embeds[:, :dh]
    embB = embeds[:, dh:]
    f32 = jnp.float32
    ones_c = jnp.ones((CHUNK,), f32)
    zer_d = jnp.zeros((IO_CH, dh), f32)
    zer_1 = jnp.zeros((IO_CH,), f32)

    y1, y2, cnt, o2 = _make_sc_spmm(n, e, d)(
        rc, embA, embB, ones_c, zer_d, zer_1)

    u = jax.random.uniform(jax.random.key(42), (n,), dtype=f32)
    scores = _score_call(embeds, y1, y2, cnt, o2, u).reshape(n)
    _, seeds = lax.top_k(scores, 1000)
    return (scores, seeds)


# speedup vs baseline: 1.0388x; 1.0388x over previous
import functools

import jax
import jax.numpy as jnp
from jax import lax
from jax.experimental import pallas as pl
from jax.experimental.pallas import tpu as pltpu
from jax.experimental.pallas import tpu_sc as plsc

NC = 2
NS = 16
CHUNK = 80
VBUF = 5
IBUF = 10
HZN = 1
IO_CH = 640


def _make_sc_spmm(n, e, d):
    dh = d // 2
    ept = e // NS
    nch = ept // CHUNK
    nround = nch // IBUF
    assert ept * NS == e and nch * CHUNK == ept and nround * IBUF == nch
    look = VBUF - HZN

    mesh = plsc.VectorSubcoreMesh(
        core_axis_name="c", subcore_axis_name="s", num_cores=NC,
        num_subcores=NS)

    def body(rc_h, embA_h, embB_h, ones_h, zerd_h, zer1_h,
             f_out, g2_out, cnt_out, o2_out,
             f_acc, g_acc, cnt_acc, o2_acc,
             ibuf, vec, sbuf, onesbuf,
             semi, semg, sems, semsg, semss):
        c = lax.axis_index("c")
        s = lax.axis_index("s")
        ibase = jnp.minimum(IO_CH * s, n - IO_CH)

        pltpu.sync_copy(zerd_h, f_acc.at[pl.ds(ibase, IO_CH), :])
        pltpu.sync_copy(zerd_h, g_acc.at[pl.ds(ibase, IO_CH), :])

        @pl.when(c == 0)
        def _():
            pltpu.sync_copy(zer1_h, cnt_acc.at[pl.ds(ibase, IO_CH)])
            pltpu.sync_copy(zer1_h, o2_acc.at[pl.ds(ibase, IO_CH)])

        pltpu.sync_copy(ones_h, onesbuf)
        plsc.subcore_barrier()

        def istart(k, slot):
            pltpu.async_copy(rc_h.at[s * nch + k], ibuf.at[slot],
                             semi.at[slot])

        def iwait(slot):
            pltpu.make_async_copy(rc_h.at[0], ibuf.at[slot],
                                  semi.at[slot]).wait()

        def run_pass(tbl, acc, scalar_tbl, scalar_acc, scalar_ones):
            do_scalar = scalar_acc is not None

            def gstart(k, bv, bi):
                pltpu.async_copy(tbl.at[ibuf.at[bi, 1]], vec.at[bv],
                                 semg.at[bv])
                if do_scalar and not scalar_ones:
                    pltpu.async_copy(scalar_tbl.at[ibuf.at[bi, 1]],
                                     sbuf.at[bv], semsg.at[bv])

            def gwait(bv):
                pltpu.make_async_copy(tbl.at[ibuf.at[0, 1]], vec.at[bv],
                                      semg.at[bv]).wait()
                if do_scalar and not scalar_ones:
                    pltpu.make_async_copy(scalar_tbl.at[ibuf.at[0, 1]],
                                          sbuf.at[bv], semsg.at[bv]).wait()

            def sstart(k, bv, bi):
                pltpu.async_copy(vec.at[bv], acc.at[ibuf.at[bi, 0]],
                                 sems.at[bv], add=True)
                if do_scalar:
                    src = onesbuf if scalar_ones else sbuf.at[bv]
                    pltpu.async_copy(src, scalar_acc.at[ibuf.at[bi, 0]],
                                     semss.at[bv], add=True)

            def swait(bv):
                pltpu.make_async_copy(vec.at[bv], acc.at[ibuf.at[0, 0]],
                                      sems.at[bv]).wait()
                if do_scalar:
                    src = onesbuf if scalar_ones else sbuf.at[bv]
                    pltpu.make_async_copy(src, scalar_acc.at[ibuf.at[0, 0]],
                                          semss.at[bv]).wait()

            for m in range(IBUF):
                istart(m, m)
            for b in range(VBUF):
                iwait(b)
                gstart(b, b, b)

            def round_body(g, carry):
                for m in range(IBUF):
                    k = g * IBUF + m
                    bv = m % VBUF
                    gwait(bv)
                    sstart(k, bv, m)
                    j = k - HZN
                    bj = (m - HZN) % VBUF
                    ij = (m - HZN) % IBUF

                    @pl.when(j >= 0)
                    def _():
                        swait(bj)

                        @pl.when(k + look < nch)
                        def _():
                            iwait((m + look) % IBUF)
                            gstart(k + look, bj, (m + look) % IBUF)

                        @pl.when(j + IBUF < nch)
                        def _():
                            istart(j + IBUF, ij)

                return carry

            lax.fori_loop(0, nround, round_body, 0)
            for t in range(HZN):
                swait((nch - HZN + t) % VBUF)

        @pl.when(c == 0)
        def _():
            run_pass(embA_h, f_acc, None, cnt_acc, True)

        @pl.when(c == 1)
        def _():
            run_pass(embB_h, f_acc, None, None, False)

        plsc.subcore_barrier()

        @pl.when(c == 0)
        def _():
            run_pass(f_acc, g_acc, cnt_acc, o2_acc, False)

        @pl.when(c == 1)
        def _():
            run_pass(f_acc, g_acc, None, None, False)

        plsc.subcore_barrier()

        pltpu.sync_copy(f_acc.at[pl.ds(ibase, IO_CH), :],
                        f_out.at[pl.ds(ibase, IO_CH), pl.ds(dh * c, dh)])
        pltpu.sync_copy(g_acc.at[pl.ds(ibase, IO_CH), :],
                        g2_out.at[pl.ds(ibase, IO_CH), pl.ds(dh * c, dh)])

        @pl.when(c == 0)
        def _():
            pltpu.sync_copy(cnt_acc.at[pl.ds(ibase, IO_CH)],
                            cnt_out.at[pl.ds(ibase, IO_CH)])
            pltpu.sync_copy(o2_acc.at[pl.ds(ibase, IO_CH)],
                            o2_out.at[pl.ds(ibase, IO_CH)])

    f32 = jnp.float32
    return pl.kernel(
        body,
        out_type=(
            jax.ShapeDtypeStruct((n, d), f32),
            jax.ShapeDtypeStruct((n, d), f32),
            jax.ShapeDtypeStruct((n,), f32),
            jax.ShapeDtypeStruct((n,), f32),
        ),
        mesh=mesh,
        compiler_params=pltpu.CompilerParams(use_tc_tiling_on_sc=False),
        scratch_types=[
            pltpu.VMEM_SHARED((n, dh), f32),
            pltpu.VMEM_SHARED((n, dh), f32),
            pltpu.VMEM_SHARED((n,), f32),
            pltpu.VMEM_SHARED((n,), f32),
            pltpu.VMEM((IBUF, 2, CHUNK), jnp.int32),
            pltpu.VMEM((VBUF, CHUNK, dh), f32),
            pltpu.VMEM((VBUF, CHUNK), f32),
            pltpu.VMEM((CHUNK,), f32),
            pltpu.SemaphoreType.DMA((IBUF,)),
            pltpu.SemaphoreType.DMA((VBUF,)),
            pltpu.SemaphoreType.DMA((VBUF,)),
            pltpu.SemaphoreType.DMA((VBUF,)),
            pltpu.SemaphoreType.DMA((VBUF,)),
        ],
    )


def _score_body(e_ref, y1_ref, y2_ref, c_ref, q_ref, u_ref, out_ref):
    emb = e_ref[...]
    y1 = y1_ref[...]
    y2 = y2_ref[...]
    c = c_ref[...]
    q = q_ref[...]
    u = u_ref[...]
    f = y1 - emb
    scd = (y2 - y1) - f - c * emb
    num = f + scd
    scd_num = q - c - c
    den = c + scd_num + 1e-8
    sub = num / den
    nrm = jnp.sqrt(jnp.sum(sub * sub, axis=-1, keepdims=True))
    sub = sub / jnp.maximum(nrm, 1e-12)
    enrm = jnp.sqrt(jnp.sum(emb * emb, axis=-1, keepdims=True))
    embn = emb / jnp.maximum(enrm, 1e-12)
    dot = jnp.sum(sub * embn, axis=-1, keepdims=True)
    sig = jax.nn.sigmoid(dot)
    uu = jnp.where(u == 0, 1e-8, u)
    noise = -jnp.log(-jnp.log(uu))
    out_ref[...] = jnp.log(sig) + noise


def _score_call(emb, f, g2, cnt, o2, u):
    n, d = emb.shape
    br = 2000
    grid = (n // br,)
    rspec = pl.BlockSpec((br, d), lambda i: (i, 0))
    cspec = pl.BlockSpec((br, 1), lambda i: (i, 0))
    return pl.pallas_call(
        _score_body,
        grid=grid,
        in_specs=[rspec, rspec, rspec, cspec, cspec, cspec],
        out_specs=cspec,
        out_shape=jax.ShapeDtypeStruct((n, 1), jnp.float32),
    )(emb, f, g2, cnt.reshape(n, 1), o2.reshape(n, 1), u.reshape(n, 1))


def kernel(allOneAdj, embeds):
    n, d = embeds.shape
    e = allOneAdj.shape[1]
    dh = d // 2
    rc = jnp.stack([allOneAdj[0].reshape(e // CHUNK, CHUNK),
                    allOneAdj[1].reshape(e // CHUNK, CHUNK)], axis=1)
    embA = embeds[:, :dh]
    embB = embeds[:, dh:]
    f32 = jnp.float32
    ones_c = jnp.ones((CHUNK,), f32)
    zer_d = jnp.zeros((IO_CH, dh), f32)
    zer_1 = jnp.zeros((IO_CH,), f32)

    y1, y2, cnt, o2 = _make_sc_spmm(n, e, d)(
        rc, embA, embB, ones_c, zer_d, zer_1)

    u = jax.random.uniform(jax.random.key(42), (n,), dtype=f32)
    scores = _score_call(embeds, y1, y2, cnt, o2, u).reshape(n)
    _, seeds = lax.approx_max_k(scores, 1000, recall_target=1.0)
    seeds = seeds.astype(jnp.int32)
    return (scores, seeds)
